# baseline (speedup 1.0000x reference)
import functools

import jax
import jax.numpy as jnp
from jax import lax
from jax.experimental import pallas as pl
from jax.experimental.pallas import tpu as pltpu
from jax.experimental.pallas import tpu_sc as plsc

VOCAB = 1000000
EMBED_DIM = 32
BATCH = 16384
FIELDS = 26
TOTAL = BATCH * FIELDS

NUM_CORES = 2
NUM_SUBCORES = 16
NUM_WORKERS = NUM_CORES * NUM_SUBCORES
PER_WORKER = TOTAL // NUM_WORKERS
CHUNK = 1024
NUM_CHUNKS = PER_WORKER // CHUNK
NBUF = 3

TC_BC = 8192
TC_GRID = (VOCAB + TC_BC - 1) // TC_BC

assert PER_WORKER * NUM_WORKERS == TOTAL
assert CHUNK * NUM_CHUNKS == PER_WORKER


def _tc_transpose_body(emb_t_ref, out_ref):
    blk = emb_t_ref[...]
    t = lax.dot_general(
        blk,
        jnp.eye(EMBED_DIM, dtype=jnp.float32),
        (((0,), (0,)), ((), ())),
        precision=lax.Precision.HIGHEST,
        preferred_element_type=jnp.float32,
    )
    t = t.reshape(TC_BC // 4, 4, EMBED_DIM)
    packed = jnp.concatenate([t[:, r, :] for r in range(4)], axis=1)
    out_ref[...] = packed.reshape(TC_BC * EMBED_DIM)


_tc_transpose = pl.pallas_call(
    _tc_transpose_body,
    grid=(TC_GRID,),
    in_specs=[
        pl.BlockSpec((EMBED_DIM, TC_BC), lambda i: (0, i)),
    ],
    out_specs=pl.BlockSpec((TC_BC * EMBED_DIM,), lambda i: (i,)),
    out_shape=jax.ShapeDtypeStruct((VOCAB * EMBED_DIM,), jnp.float32),
)


@functools.partial(
    pl.kernel,
    out_type=jax.ShapeDtypeStruct((TOTAL, EMBED_DIM), jnp.float32),
    mesh=plsc.VectorSubcoreMesh(core_axis_name="c", subcore_axis_name="s"),
    scratch_types=[
        [pltpu.VMEM((CHUNK,), jnp.int32) for _ in range(NBUF)],
        [pltpu.VMEM((CHUNK, EMBED_DIM), jnp.float32) for _ in range(NBUF)],
        [pltpu.SemaphoreType.DMA for _ in range(NBUF)],
        [pltpu.SemaphoreType.DMA for _ in range(NBUF)],
    ],
    compiler_params=pltpu.CompilerParams(use_tc_tiling_on_sc=False),
)
def _gather_rows(table_hbm, idx_hbm, out_hbm, idx_v, rows_v, gsem, ssem):
    wid = lax.axis_index("s") * NUM_CORES + lax.axis_index("c")
    base = wid * PER_WORKER

    gathers = [None] * NUM_CHUNKS
    stores = [None] * NUM_CHUNKS

    def start_gather(i):
        b = i % NBUF
        off = pl.multiple_of(base + i * CHUNK, 8)
        pltpu.sync_copy(idx_hbm.at[pl.ds(off, CHUNK)], idx_v[b])
        gathers[i] = pltpu.async_copy(table_hbm.at[idx_v[b]], rows_v[b], gsem[b])

    start_gather(0)
    start_gather(1)
    for i in range(NUM_CHUNKS):
        b = i % NBUF
        j = i + 2
        if j < NUM_CHUNKS:
            if j - NBUF >= 0:
                stores[j - NBUF].wait()
            start_gather(j)
        gathers[i].wait()
        off = pl.multiple_of(base + i * CHUNK, 8)
        stores[i] = pltpu.async_copy(rows_v[b], out_hbm.at[pl.ds(off, CHUNK)], ssem[b])
    for i in range(max(0, NUM_CHUNKS - NBUF), NUM_CHUNKS):
        stores[i].wait()


def kernel(embedding, x):
    emb_t = embedding.T
    table = _tc_transpose(emb_t).reshape(VOCAB, EMBED_DIM)
    idx = x.reshape(TOTAL).astype(jnp.int32)
    out = _gather_rows(table, idx)
    return out.reshape(BATCH, FIELDS, EMBED_DIM)


# speedup vs baseline: 1.4065x; 1.4065x over previous
import functools

import jax
import jax.numpy as jnp
from jax import lax
from jax.experimental import pallas as pl
from jax.experimental.pallas import tpu as pltpu
from jax.experimental.pallas import tpu_sc as plsc

VOCAB = 1000000
EMBED_DIM = 32
BATCH = 16384
FIELDS = 26
TOTAL = BATCH * FIELDS

NUM_CORES = 2
NUM_SUBCORES = 16
NUM_WORKERS = NUM_CORES * NUM_SUBCORES
PER_WORKER = TOTAL // NUM_WORKERS
CHUNK = 1024
NUM_CHUNKS = PER_WORKER // CHUNK
NBUF = 3

TC_BC = 8192
TC_GRID = (VOCAB + TC_BC - 1) // TC_BC

assert PER_WORKER * NUM_WORKERS == TOTAL
assert CHUNK * NUM_CHUNKS == PER_WORKER


def _tc_transpose_body(emb_t_ref, out_ref):
    blk = emb_t_ref[...]
    t = jnp.transpose(blk).reshape(TC_BC // 4, 4, EMBED_DIM)
    packed = jnp.concatenate([t[:, r, :] for r in range(4)], axis=1)
    out_ref[...] = packed.reshape(TC_BC * EMBED_DIM)


_tc_transpose = pl.pallas_call(
    _tc_transpose_body,
    grid=(TC_GRID,),
    in_specs=[
        pl.BlockSpec((EMBED_DIM, TC_BC), lambda i: (0, i)),
    ],
    out_specs=pl.BlockSpec((TC_BC * EMBED_DIM,), lambda i: (i,)),
    out_shape=jax.ShapeDtypeStruct((VOCAB * EMBED_DIM,), jnp.float32),
)


@functools.partial(
    pl.kernel,
    out_type=jax.ShapeDtypeStruct((TOTAL, EMBED_DIM), jnp.float32),
    mesh=plsc.VectorSubcoreMesh(core_axis_name="c", subcore_axis_name="s"),
    scratch_types=[
        [pltpu.VMEM((CHUNK,), jnp.int32) for _ in range(NBUF)],
        [pltpu.VMEM((CHUNK, EMBED_DIM), jnp.float32) for _ in range(NBUF)],
        [pltpu.SemaphoreType.DMA for _ in range(NBUF)],
        [pltpu.SemaphoreType.DMA for _ in range(NBUF)],
    ],
    compiler_params=pltpu.CompilerParams(use_tc_tiling_on_sc=False),
)
def _gather_rows(table_hbm, idx_hbm, out_hbm, idx_v, rows_v, gsem, ssem):
    wid = lax.axis_index("s") * NUM_CORES + lax.axis_index("c")
    base = wid * PER_WORKER

    gathers = [None] * NUM_CHUNKS
    stores = [None] * NUM_CHUNKS

    def start_gather(i):
        b = i % NBUF
        off = pl.multiple_of(base + i * CHUNK, 8)
        pltpu.sync_copy(idx_hbm.at[pl.ds(off, CHUNK)], idx_v[b])
        gathers[i] = pltpu.async_copy(table_hbm.at[idx_v[b]], rows_v[b], gsem[b])

    start_gather(0)
    start_gather(1)
    for i in range(NUM_CHUNKS):
        b = i % NBUF
        j = i + 2
        if j < NUM_CHUNKS:
            if j - NBUF >= 0:
                stores[j - NBUF].wait()
            start_gather(j)
        gathers[i].wait()
        off = pl.multiple_of(base + i * CHUNK, 8)
        stores[i] = pltpu.async_copy(rows_v[b], out_hbm.at[pl.ds(off, CHUNK)], ssem[b])
    for i in range(max(0, NUM_CHUNKS - NBUF), NUM_CHUNKS):
        stores[i].wait()


def kernel(embedding, x):
    emb_t = embedding.T
    table = _tc_transpose(emb_t).reshape(VOCAB, EMBED_DIM)
    idx = x.reshape(TOTAL).astype(jnp.int32)
    out = _gather_rows(table, idx)
    return out.reshape(BATCH, FIELDS, EMBED_DIM)


# speedup vs baseline: 1.4144x; 1.0057x over previous
import functools

import jax
import jax.numpy as jnp
from jax import lax
from jax.experimental import pallas as pl
from jax.experimental.pallas import tpu as pltpu
from jax.experimental.pallas import tpu_sc as plsc

VOCAB = 1000000
EMBED_DIM = 32
BATCH = 16384
FIELDS = 26
TOTAL = BATCH * FIELDS

NUM_CORES = 2
NUM_SUBCORES = 16
NUM_WORKERS = NUM_CORES * NUM_SUBCORES
PER_WORKER = TOTAL // NUM_WORKERS
CHUNK = 1024
NUM_CHUNKS = PER_WORKER // CHUNK
NBUF = 3

TC_BC = 16384
TC_GRID = (VOCAB + TC_BC - 1) // TC_BC

assert PER_WORKER * NUM_WORKERS == TOTAL
assert CHUNK * NUM_CHUNKS == PER_WORKER


def _tc_transpose_body(emb_t_ref, out_ref):
    blk = emb_t_ref[...]
    t = jnp.transpose(blk).reshape(TC_BC // 4, 4, EMBED_DIM)
    packed = jnp.concatenate([t[:, r, :] for r in range(4)], axis=1)
    out_ref[...] = packed.reshape(TC_BC * EMBED_DIM)


_tc_transpose = pl.pallas_call(
    _tc_transpose_body,
    grid=(TC_GRID,),
    in_specs=[
        pl.BlockSpec((EMBED_DIM, TC_BC), lambda i: (0, i)),
    ],
    out_specs=pl.BlockSpec((TC_BC * EMBED_DIM,), lambda i: (i,)),
    out_shape=jax.ShapeDtypeStruct((VOCAB * EMBED_DIM,), jnp.float32),
)


@functools.partial(
    pl.kernel,
    out_type=jax.ShapeDtypeStruct((TOTAL, EMBED_DIM), jnp.float32),
    mesh=plsc.VectorSubcoreMesh(core_axis_name="c", subcore_axis_name="s"),
    scratch_types=[
        [pltpu.VMEM((CHUNK,), jnp.int32) for _ in range(NBUF)],
        [pltpu.VMEM((CHUNK, EMBED_DIM), jnp.float32) for _ in range(NBUF)],
        [pltpu.SemaphoreType.DMA for _ in range(NBUF)],
        [pltpu.SemaphoreType.DMA for _ in range(NBUF)],
    ],
    compiler_params=pltpu.CompilerParams(use_tc_tiling_on_sc=False),
)
def _gather_rows(table_hbm, idx_hbm, out_hbm, idx_v, rows_v, gsem, ssem):
    wid = lax.axis_index("s") * NUM_CORES + lax.axis_index("c")
    base = wid * PER_WORKER

    gathers = [None] * NUM_CHUNKS
    stores = [None] * NUM_CHUNKS

    def start_gather(i):
        b = i % NBUF
        off = pl.multiple_of(base + i * CHUNK, 8)
        pltpu.sync_copy(idx_hbm.at[pl.ds(off, CHUNK)], idx_v[b])
        gathers[i] = pltpu.async_copy(table_hbm.at[idx_v[b]], rows_v[b], gsem[b])

    start_gather(0)
    start_gather(1)
    for i in range(NUM_CHUNKS):
        b = i % NBUF
        j = i + 2
        if j < NUM_CHUNKS:
            if j - NBUF >= 0:
                stores[j - NBUF].wait()
            start_gather(j)
        gathers[i].wait()
        off = pl.multiple_of(base + i * CHUNK, 8)
        stores[i] = pltpu.async_copy(rows_v[b], out_hbm.at[pl.ds(off, CHUNK)], ssem[b])
    for i in range(max(0, NUM_CHUNKS - NBUF), NUM_CHUNKS):
        stores[i].wait()


def kernel(embedding, x):
    emb_t = embedding.T
    table = _tc_transpose(emb_t).reshape(VOCAB, EMBED_DIM)
    idx = x.reshape(TOTAL).astype(jnp.int32)
    out = _gather_rows(table, idx)
    return out.reshape(BATCH, FIELDS, EMBED_DIM)


# speedup vs baseline: 1.4204x; 1.0042x over previous
import functools

import jax
import jax.numpy as jnp
from jax import lax
from jax.experimental import pallas as pl
from jax.experimental.pallas import tpu as pltpu
from jax.experimental.pallas import tpu_sc as plsc

VOCAB = 1000000
EMBED_DIM = 32
BATCH = 16384
FIELDS = 26
TOTAL = BATCH * FIELDS

NUM_CORES = 2
NUM_SUBCORES = 16
NUM_WORKERS = NUM_CORES * NUM_SUBCORES
PER_WORKER = TOTAL // NUM_WORKERS
CHUNK = 1024
NUM_CHUNKS = PER_WORKER // CHUNK
NBUF = 3

TC_BC = 32768
TC_GRID = (VOCAB + TC_BC - 1) // TC_BC

assert PER_WORKER * NUM_WORKERS == TOTAL
assert CHUNK * NUM_CHUNKS == PER_WORKER


def _tc_transpose_body(emb_t_ref, out_ref):
    blk = emb_t_ref[...]
    t = jnp.transpose(blk).reshape(TC_BC // 4, 4, EMBED_DIM)
    packed = jnp.concatenate([t[:, r, :] for r in range(4)], axis=1)
    out_ref[...] = packed.reshape(TC_BC * EMBED_DIM)


_tc_transpose = pl.pallas_call(
    _tc_transpose_body,
    grid=(TC_GRID,),
    in_specs=[
        pl.BlockSpec((EMBED_DIM, TC_BC), lambda i: (0, i)),
    ],
    out_specs=pl.BlockSpec((TC_BC * EMBED_DIM,), lambda i: (i,)),
    out_shape=jax.ShapeDtypeStruct((VOCAB * EMBED_DIM,), jnp.float32),
)


@functools.partial(
    pl.kernel,
    out_type=jax.ShapeDtypeStruct((TOTAL, EMBED_DIM), jnp.float32),
    mesh=plsc.VectorSubcoreMesh(core_axis_name="c", subcore_axis_name="s"),
    scratch_types=[
        [pltpu.VMEM((CHUNK,), jnp.int32) for _ in range(NBUF)],
        [pltpu.VMEM((CHUNK, EMBED_DIM), jnp.float32) for _ in range(NBUF)],
        [pltpu.SemaphoreType.DMA for _ in range(NBUF)],
        [pltpu.SemaphoreType.DMA for _ in range(NBUF)],
    ],
    compiler_params=pltpu.CompilerParams(use_tc_tiling_on_sc=False),
)
def _gather_rows(table_hbm, idx_hbm, out_hbm, idx_v, rows_v, gsem, ssem):
    wid = lax.axis_index("s") * NUM_CORES + lax.axis_index("c")
    base = wid * PER_WORKER

    gathers = [None] * NUM_CHUNKS
    stores = [None] * NUM_CHUNKS

    def start_gather(i):
        b = i % NBUF
        off = pl.multiple_of(base + i * CHUNK, 8)
        pltpu.sync_copy(idx_hbm.at[pl.ds(off, CHUNK)], idx_v[b])
        gathers[i] = pltpu.async_copy(table_hbm.at[idx_v[b]], rows_v[b], gsem[b])

    start_gather(0)
    start_gather(1)
    for i in range(NUM_CHUNKS):
        b = i % NBUF
        j = i + 2
        if j < NUM_CHUNKS:
            if j - NBUF >= 0:
                stores[j - NBUF].wait()
            start_gather(j)
        gathers[i].wait()
        off = pl.multiple_of(base + i * CHUNK, 8)
        stores[i] = pltpu.async_copy(rows_v[b], out_hbm.at[pl.ds(off, CHUNK)], ssem[b])
    for i in range(max(0, NUM_CHUNKS - NBUF), NUM_CHUNKS):
        stores[i].wait()


def kernel(embedding, x):
    emb_t = embedding.T
    table = _tc_transpose(emb_t).reshape(VOCAB, EMBED_DIM)
    idx = x.reshape(TOTAL).astype(jnp.int32)
    out = _gather_rows(table, idx)
    return out.reshape(BATCH, FIELDS, EMBED_DIM)
